# baseline (speedup 1.0000x reference)
import jax
import jax.numpy as jnp
from jax import lax
from jax.experimental import pallas as pl
from jax.experimental.pallas import tpu as pltpu
from jax.experimental.pallas import tpu_sc as plsc

_N = 10000
_NP = 10240
_E = 320000
_DIN, _DH, _DO = 128, 64, 32
_NT = 16
_RPT = _N // _NT
_SEG = _NP // _NT
_CHK = 80
_NCH_G = (_E // _NT) // _CHK
_NCH_C = (_E // (2 * _NT)) // _CHK
_BLK = 2000
_ZPERM = sum(([32 * c + k, 32 * c + 16 + k] for c in range(2)
              for k in range(16)), [])
_BLKP = 2048


def _elu(x):
    return jnp.where(x > 0, x, jnp.exp(jnp.minimum(x, 0.0)) - 1.0)


def _prepass_body(f1, f2, f3, w1, w2, w3, wp1, wp2, wp3, a1, a2, a3,
                  z1, z2, z3, zs1, zd1, zs2, zd2, zs3, zd3):
    for f, w, wp, a, z, zs, zd in ((f1, w1, wp1, a1, z1, zs1, zd1),
                                   (f2, w2, wp2, a2, z2, zs2, zd2),
                                   (f3, w3, wp3, a3, z3, zs3, zd3)):
        zb = jnp.dot(f[...], w[...], preferred_element_type=jnp.float32)
        zp = jnp.dot(f[...], wp[...], preferred_element_type=jnp.float32)
        z[...] = zp.astype(jnp.bfloat16)
        sd = jnp.dot(zb, a[...], preferred_element_type=jnp.float32)
        zs[...] = sd[:, 0:1]
        zd[...] = sd[:, 1:2]


def _h2_body(h1, w2, out):
    h = _elu(h1[0])
    out[0] = jnp.dot(h, w2[0], preferred_element_type=jnp.float32)


def _fuse_body(h2b, sums, cnts, w_om, u_om,
               d1W1, d1b1, d2W1, d2b1, d1W2, d1b2, d2W2, d2b2,
               d1W3, d1b3, d2W3, d2b3,
               emb_o, r1_o, r2_o, r3_o, hp1_o, hp2_o, hp3_o):
    h = (h2b[0], h2b[1], h2b[2])
    ex = []
    for c in range(3):
        v = jnp.tanh(jnp.dot(h[c], w_om[...], preferred_element_type=jnp.float32))
        vu = jnp.dot(v, u_om[...], preferred_element_type=jnp.float32)
        ex.append(jnp.exp(vu))
    denom = ex[0] + ex[1] + ex[2]
    emb = (h[0] * (ex[0] / denom) + h[1] * (ex[1] / denom)
           + h[2] * (ex[2] / denom))
    emb_o[...] = emb
    for dw1, db1, dw2, db2, ro in ((d1W1, d1b1, d2W1, d2b1, r1_o),
                                   (d1W2, d1b2, d2W2, d2b2, r2_o),
                                   (d1W3, d1b3, d2W3, d2b3, r3_o)):
        t = _elu(jnp.dot(emb, dw1[...], preferred_element_type=jnp.float32)
                       + db1[...])
        ro[...] = (jnp.dot(t, dw2[...], preferred_element_type=jnp.float32)
                   + db2[...])
    cnt3 = jnp.sum(cnts[...], axis=0).reshape(3, -1)
    s_all = sums[...]
    for m, hpo in enumerate((hp1_o, hp2_o, hp3_o)):
        sm = s_all[0, m] + s_all[1, m]
        hpo[...] = sm / jnp.maximum(cnt3[m], 1.0)[:, None]


def _zero_1d(ref, n):
    @pl.loop(0, n // 16)
    def _(i):
        ref[pl.ds(i * 16, 16)] = jnp.zeros((16,), jnp.float32)


def _zero_rows(ref, nrows, width):
    @pl.loop(0, nrows)
    def _(r):
        for c in range(width // 16):
            ref[r, pl.ds(c * 16, 16)] = jnp.zeros((16,), jnp.float32)


def _copy_rows_out(src_sh, dst, base):
    for q in range(_SEG // _CHK):
        pltpu.sync_copy(src_sh.at[pl.ds(base + q * _CHK, _CHK)],
                        dst.at[pl.ds(base + q * _CHK, _CHK)])


def _zero_sh_rows(zero_buf, dst_sh, base, sem=None):
    if sem is None:
        for q in range(_SEG // _CHK):
            pltpu.sync_copy(zero_buf, dst_sh.at[pl.ds(base + q * _CHK, _CHK)])
    else:
        for q in range(_SEG // _CHK):
            pltpu.async_copy(zero_buf, dst_sh.at[pl.ds(base + q * _CHK, _CHK)],
                             sem)
        for q in range(_SEG // _CHK):
            pltpu.make_async_copy(
                zero_buf, dst_sh.at[pl.ds(base + q * _CHK, _CHK)], sem).wait()


_SROW = _NP // 16
_SRQ = _SROW // 128


def _gat_body(ei1, ei2, ei3, eiC, zs1, zd1, zs2, zd2, zs3, zd3, z1, z2, z3,
              h1_out,
              src2, dst2, zs_t, zd_t, s_loc, rowsA, rowsB, rowbfA, rowbfB,
              alpha_b, iota_r, zrow_b, sdiv, s_sum_sh, h1_sh,
              gsA, gsB, ssA, ssB):
    cid = lax.axis_index("c")
    sid = lax.axis_index("s")
    for q in range(_SRQ):
        for g in range(8):
            iota_r[q, pl.ds(g * 16, 16)] = (
                lax.iota(jnp.int32, 16) + (q * 128 + g * 16))
    _zero_rows(zrow_b, _SROW // _NT, 16)
    for rr in (_NCH_G, _NCH_G + 1):
        for g in range(_CHK // 16):
            src2[rr, pl.ds(g * 16, 16)] = jnp.zeros((16,), jnp.int32)
    tabs = ((zs1, zd1, z1, ei1), (zs2, zd2, z2, ei2), (zs3, zd3, z3, ei3))
    for a in range(3):
        zs_h, zd_h, z_h, ei_h = tabs[a]

        @pl.when(cid == 0)
        def _():
            pltpu.sync_copy(ei_h.at[0, sid], src2.at[pl.ds(0, _NCH_G)])
            pltpu.sync_copy(ei_h.at[1, sid], dst2)

        if a == 0:
            @pl.when(cid == 1)
            def _():
                pltpu.sync_copy(eiC.at[0, sid], src2.at[pl.ds(0, _NCH_G)])
                pltpu.sync_copy(eiC.at[1, sid], dst2)

        pltpu.sync_copy(zs_h, zs_t)
        pltpu.sync_copy(zd_h, zd_t)
        _zero_rows(s_loc, _SROW, 16)
        _zero_rows(rowsA, _CHK, _DH)
        pltpu.sync_copy(zrow_b, s_sum_sh.at[pl.ds((_SROW // _NT) * sid,
                                                  _SROW // _NT)])
        _zero_sh_rows(rowsA, h1_sh, sid * _SEG, ssA)
        plsc.subcore_barrier()

        @pl.loop(0, _NCH_G, unroll=2)
        def _(j):
            for k in range(_CHK // 16):
                sv = src2[j, pl.ds(k * 16, 16)]
                dv = dst2[j, pl.ds(k * 16, 16)]
                e = plsc.load_gather(zs_t, [sv]) + plsc.load_gather(zd_t, [dv])
                e = jnp.maximum(e, 0.2 * e)
                exv = jnp.exp(e)
                plsc.addupdate_scatter(
                    s_loc, [lax.shift_right_logical(dv, 4),
                            lax.bitwise_and(dv, 15)], exv)

        for q in range(_SRQ):
            pltpu.sync_copy(s_loc.at[pl.ds(q * 128, 128)],
                            s_sum_sh.at[iota_r.at[q]], add=True)

        bufs = (rowsA, rowsB)
        bfb = (rowbfA, rowbfB)
        gsems = (gsA, gsB)
        ssems = (ssA, ssB)

        def _fire_g(j, b):
            pltpu.async_copy(z_h.at[src2.at[j]], bfb[b], gsems[b])

        def _wait_g(j, b):
            pltpu.make_async_copy(z_h.at[src2.at[j]], bfb[b],
                                  gsems[b]).wait()

        def _fire_s(j, b):
            pltpu.async_copy(bufs[b], h1_sh.at[dst2.at[j]], ssems[b],
                             add=True)

        def _wait_s(j, b):
            pltpu.make_async_copy(bufs[b], h1_sh.at[dst2.at[j]],
                                  ssems[b]).wait()

        def _scale(j, b):
            rbf = bfb[b]
            rows = bufs[b]
            for k in range(_CHK // 16):
                sv = src2[j, pl.ds(k * 16, 16)]
                dv = dst2[j, pl.ds(k * 16, 16)]
                e = plsc.load_gather(zs_t, [sv]) + plsc.load_gather(zd_t, [dv])
                e = jnp.maximum(e, 0.2 * e)
                alpha_b[pl.ds(k * 16, 16)] = jnp.exp(e)

            @pl.loop(0, _CHK, unroll=4)
            def _(r):
                av = plsc.load_gather(alpha_b, [jnp.full((16,), r, jnp.int32)])
                for c in range(_DH // 32):
                    w = rbf[r, pl.ds(c * 32, 32)]
                    lo, hi = plsc.unpack(
                        w, format=plsc.PackFormat.INTERLEAVED,
                        preferred_element_type=jnp.float32)
                    rows[r, pl.ds(c * 32, 16)] = lo * av
                    rows[r, pl.ds(c * 32 + 16, 16)] = hi * av

        _fire_g(0, 0)
        _fire_g(1, 1)
        _wait_g(0, 0); _scale(0, 0); _fire_s(0, 0); _fire_g(2, 0)
        _wait_g(1, 1); _scale(1, 1); _fire_s(1, 1); _fire_g(3, 1)

        @pl.loop(2, _NCH_G, step=2)
        def _(t):
            for d in range(2):
                b = d
                _wait_g(t + d, b)
                _wait_s(t + d - 2, b)
                _scale(t + d, b)
                _fire_s(t + d, b)
                _fire_g(t + d + 2, b)

        _wait_g(_NCH_G, 0)
        _wait_g(_NCH_G + 1, 1)
        _wait_s(_NCH_G - 2, 0)
        _wait_s(_NCH_G - 1, 1)
        plsc.subcore_barrier()

        pltpu.sync_copy(s_sum_sh.at[pl.ds(sid * (_SEG // 16), _SEG // 16)],
                        sdiv)
        nq = _SEG // _CHK

        def _div_out(q, b):
            rows = bufs[b]
            rbase = sid * _SEG + q * _CHK

            @pl.loop(0, _CHK)
            def _(r):
                rq = r + q * _CHK
                sv = plsc.load_gather(
                    sdiv, [jnp.full((16,), lax.shift_right_logical(rq, 4),
                                    jnp.int32),
                           jnp.full((16,), lax.bitwise_and(rq, 15),
                                    jnp.int32)])
                inv = 1.0 / (sv + 1e-16)
                for c in range(_DH // 16):
                    rows[r, pl.ds(c * 16, 16)] = (
                        rows[r, pl.ds(c * 16, 16)] * inv)

            pltpu.sync_copy(rows, h1_out.at[cid, a, pl.ds(rbase, _CHK)])

        def _fire_in(q, b):
            pltpu.async_copy(h1_sh.at[pl.ds(sid * _SEG + q * _CHK, _CHK)],
                             bufs[b], gsems[b])

        def _wait_in(q, b):
            pltpu.make_async_copy(
                h1_sh.at[pl.ds(sid * _SEG + q * _CHK, _CHK)],
                bufs[b], gsems[b]).wait()

        _fire_in(0, 0)
        for q in range(nq):
            if q + 1 < nq:
                _fire_in(q + 1, (q + 1) % 2)
            _wait_in(q, q % 2)
            _div_out(q, q % 2)
        plsc.subcore_barrier()


def _csl_body(ei1, ei2, ei3, h2a,
              sums_out, cnt_out,
              gat2, sct2, cnt_loc, rows32A, rows32B, rows32C, iota_r, zrow_b,
              h2s_sh, cnt_sh,
              gsA, gsB, gsC, ssA, ssB, ssC):
    cid = lax.axis_index("c")
    sid = lax.axis_index("s")
    for q in range(_SRQ):
        for g in range(8):
            iota_r[q, pl.ds(g * 16, 16)] = (
                lax.iota(jnp.int32, 16) + (q * 128 + g * 16))
    _zero_rows(zrow_b, _SROW // _NT, 16)
    tabs = (ei1, ei2, ei3)
    for m in range(3):
        ei_h = tabs[m]
        h2_h = h2a.at[m]
        pltpu.sync_copy(ei_h.at[1, cid, sid], gat2)
        pltpu.sync_copy(ei_h.at[0, cid, sid], sct2)
        _zero_rows(cnt_loc, _SROW, 16)
        pltpu.sync_copy(zrow_b, cnt_sh.at[pl.ds((_SROW // _NT) * sid,
                                                _SROW // _NT)])

        @pl.loop(0, _NCH_C, unroll=2)
        def _(j):
            for k in range(_CHK // 16):
                sv = sct2[j, pl.ds(k * 16, 16)]
                plsc.addupdate_scatter(
                    cnt_loc, [lax.shift_right_logical(sv, 4),
                              lax.bitwise_and(sv, 15)],
                    jnp.ones((16,), jnp.float32))

        _zero_rows(rows32A, _CHK, _DO)
        _zero_sh_rows(rows32A, h2s_sh, sid * _SEG, ssA)
        plsc.subcore_barrier()
        for q in range(_SRQ):
            pltpu.sync_copy(cnt_loc.at[pl.ds(q * 128, 128)],
                            cnt_sh.at[iota_r.at[q]], add=True)

        bufs = (rows32A, rows32B, rows32C)
        gsems = (gsA, gsB, gsC)
        ssems = (ssA, ssB, ssC)

        def _fire_g(j, b):
            pltpu.async_copy(h2_h.at[gat2.at[j]], bufs[b], gsems[b])

        def _wait_g(j, b):
            pltpu.make_async_copy(h2_h.at[gat2.at[j]], bufs[b],
                                  gsems[b]).wait()

        def _fire_s(j, b):
            pltpu.async_copy(bufs[b], h2s_sh.at[sct2.at[j]], ssems[b],
                             add=True)

        def _wait_s(j, b):
            pltpu.make_async_copy(bufs[b], h2s_sh.at[sct2.at[j]],
                                  ssems[b]).wait()

        _fire_g(0, 0)
        _fire_g(1, 1)
        _wait_g(0, 0); _fire_s(0, 0); _fire_g(2, 2)
        _wait_g(1, 1); _fire_s(1, 1); _wait_s(0, 0); _fire_g(3, 0)

        @pl.loop(2, _NCH_C - 3, step=3)
        def _(t):
            for d in range(3):
                b = (2 + d) % 3
                _wait_g(t + d, b); _fire_s(t + d, b)
                _wait_s(t + d - 1, (b + 2) % 3)
                _fire_g(t + d + 2, (b + 2) % 3)

        _wait_g(122, 2); _fire_s(122, 2); _wait_s(121, 1); _fire_g(124, 1)
        _wait_g(123, 0); _fire_s(123, 0); _wait_s(122, 2)
        _wait_g(124, 1); _fire_s(124, 1); _wait_s(123, 0)
        _wait_s(124, 1)
        plsc.subcore_barrier()
        _copy_rows_out(h2s_sh, sums_out.at[cid, m], sid * _SEG)
        pltpu.sync_copy(cnt_sh.at[pl.ds((_SROW // _NT) * sid, _SROW // _NT)],
                        cnt_out.at[cid, m, pl.ds((_SROW // _NT) * sid,
                                                 _SROW // _NT)])
        plsc.subcore_barrier()


def kernel(features_1, features_2, features_3,
           edge_index_1, edge_index_2, edge_index_3, edge_CSL,
           W1_1, as_1, ad_1, W2_1, W1_2, as_2, ad_2, W2_2,
           W1_3, as_3, ad_3, W2_3, w_omega, u_omega,
           d1W_1, d1b_1, d2W_1, d2b_1, d1W_2, d1b_2, d2W_2, d2b_2,
           d1W_3, d1b_3, d2W_3, d2b_3):
    f32 = jnp.float32
    A1 = jnp.stack([as_1, ad_1], axis=1)
    A2 = jnp.stack([as_2, ad_2], axis=1)
    A3 = jnp.stack([as_3, ad_3], axis=1)

    nb = _N // _BLK
    outs = pl.pallas_call(
        _prepass_body,
        grid=(nb,),
        in_specs=[pl.BlockSpec((_BLK, _DIN), lambda i: (i, 0))] * 3
        + [pl.BlockSpec((_DIN, _DH), lambda i: (0, 0))] * 6
        + [pl.BlockSpec((_DH, 2), lambda i: (0, 0))] * 3,
        out_specs=[pl.BlockSpec((_BLK, _DH), lambda i: (i, 0))] * 3
        + [pl.BlockSpec((_BLK, 1), lambda i: (i, 0))] * 6,
        out_shape=[jax.ShapeDtypeStruct((_N, _DH), jnp.bfloat16)] * 3
        + [jax.ShapeDtypeStruct((_N, 1), f32)] * 6,
    )(features_1, features_2, features_3, W1_1, W1_2, W1_3,
      W1_1[:, _ZPERM], W1_2[:, _ZPERM], W1_3[:, _ZPERM], A1, A2, A3)
    z1, z2, z3, zs1, zd1, zs2, zd2, zs3, zd3 = outs
    zs1, zd1, zs2, zd2, zs3, zd3 = (x.reshape(_N) for x in
                                    (zs1, zd1, zs2, zd2, zs3, zd3))

    eg = lambda e: e.reshape(2, _NT, _NCH_G, _CHK)
    ei1g, ei2g, ei3g, eiCg = (eg(e) for e in (edge_index_1, edge_index_2,
                                              edge_index_3, edge_CSL))
    h1_all = pl.kernel(
        _gat_body,
        out_type=jax.ShapeDtypeStruct((2, 3, _NP, _DH), f32),
        mesh=plsc.VectorSubcoreMesh(core_axis_name="c", subcore_axis_name="s"),
        compiler_params=pltpu.CompilerParams(needs_layout_passes=False, use_tc_tiling_on_sc=False),
        scratch_types=[
            pltpu.VMEM((_NCH_G + 2, _CHK), jnp.int32),
            pltpu.VMEM((_NCH_G, _CHK), jnp.int32),
            pltpu.VMEM((_N,), f32),
            pltpu.VMEM((_N,), f32),
            pltpu.VMEM((_SROW, 16), f32),
            pltpu.VMEM((_CHK, _DH), f32),
            pltpu.VMEM((_CHK, _DH), f32),
            pltpu.VMEM((_CHK, _DH), jnp.bfloat16),
            pltpu.VMEM((_CHK, _DH), jnp.bfloat16),
            pltpu.VMEM((_CHK,), f32),
            pltpu.VMEM((_SRQ, 128), jnp.int32),
            pltpu.VMEM((_SROW // _NT, 16), f32),
            pltpu.VMEM((_SEG // 16, 16), f32),
            pltpu.VMEM_SHARED((_SROW, 16), f32),
            pltpu.VMEM_SHARED((_NP, _DH), f32),
        ] + [pltpu.SemaphoreType.DMA] * 4,
    )(ei1g, ei2g, ei3g, eiCg, zs1, zd1, zs2, zd2, zs3, zd3, z1, z2, z3)

    h1_flat = h1_all.reshape(6, _NP, _DH)
    W2s = jnp.stack([W2_1, W2_2, W2_3, W2_1, W2_2, W2_3])
    nbp = _NP // _BLKP
    h2_all = pl.pallas_call(
        _h2_body,
        grid=(6, nbp),
        in_specs=[pl.BlockSpec((1, _BLKP, _DH), lambda m, i: (m, i, 0)),
                  pl.BlockSpec((1, _DH, _DO), lambda m, i: (m, 0, 0))],
        out_specs=pl.BlockSpec((1, _BLKP, _DO), lambda m, i: (m, i, 0)),
        out_shape=jax.ShapeDtypeStruct((6, _NP, _DO), f32),
    )(h1_flat, W2s)

    ec = lambda e: e.reshape(2, 2, _NT, _NCH_C, _CHK)
    ei1c, ei2c, ei3c = (ec(e) for e in (edge_index_1, edge_index_2,
                                        edge_index_3))
    sums, cnts = pl.kernel(
        _csl_body,
        out_type=(jax.ShapeDtypeStruct((2, 3, _NP, _DO), f32),
                  jax.ShapeDtypeStruct((2, 3, _SROW, 16), f32)),
        mesh=plsc.VectorSubcoreMesh(core_axis_name="c", subcore_axis_name="s"),
        compiler_params=pltpu.CompilerParams(needs_layout_passes=False, use_tc_tiling_on_sc=False),
        scratch_types=[
            pltpu.VMEM((_NCH_C, _CHK), jnp.int32),
            pltpu.VMEM((_NCH_C, _CHK), jnp.int32),
            pltpu.VMEM((_SROW, 16), f32),
            pltpu.VMEM((_CHK, _DO), f32),
            pltpu.VMEM((_CHK, _DO), f32),
            pltpu.VMEM((_CHK, _DO), f32),
            pltpu.VMEM((_SRQ, 128), jnp.int32),
            pltpu.VMEM((_SROW // _NT, 16), f32),
            pltpu.VMEM_SHARED((_NP, _DO), f32),
            pltpu.VMEM_SHARED((_SROW, 16), f32),
        ] + [pltpu.SemaphoreType.DMA] * 6 + [
        ],
    )(ei1c, ei2c, ei3c, h2_all)

    b1_1, b2_1 = d1b_1.reshape(1, _DH), d2b_1.reshape(1, _DIN)
    b1_2, b2_2 = d1b_2.reshape(1, _DH), d2b_2.reshape(1, _DIN)
    b1_3, b2_3 = d1b_3.reshape(1, _DH), d2b_3.reshape(1, _DIN)
    u_om = u_omega.reshape(_DO, 1)
    fuse_outs = pl.pallas_call(
        _fuse_body,
        grid=(nbp,),
        in_specs=[pl.BlockSpec((3, _BLKP, _DO), lambda i: (0, i, 0)),
                  pl.BlockSpec((2, 3, _BLKP, _DO), lambda i: (0, 0, i, 0)),
                  pl.BlockSpec((2, 3, _BLKP // 16, 16),
                               lambda i: (0, 0, i, 0)),
                  pl.BlockSpec((_DO, _DO), lambda i: (0, 0)),
                  pl.BlockSpec((_DO, 1), lambda i: (0, 0))]
        + [pl.BlockSpec((_DO, _DH), lambda i: (0, 0)),
           pl.BlockSpec((1, _DH), lambda i: (0, 0)),
           pl.BlockSpec((_DH, _DIN), lambda i: (0, 0)),
           pl.BlockSpec((1, _DIN), lambda i: (0, 0))] * 3,
        out_specs=[pl.BlockSpec((_BLKP, _DO), lambda i: (i, 0))]
        + [pl.BlockSpec((_BLKP, _DIN), lambda i: (i, 0))] * 3
        + [pl.BlockSpec((_BLKP, _DO), lambda i: (i, 0))] * 3,
        out_shape=[jax.ShapeDtypeStruct((_NP, _DO), f32)]
        + [jax.ShapeDtypeStruct((_NP, _DIN), f32)] * 3
        + [jax.ShapeDtypeStruct((_NP, _DO), f32)] * 3,
    )(h2_all, sums, cnts, w_omega, u_om,
      d1W_1, b1_1, d2W_1, b2_1, d1W_2, b1_2, d2W_2, b2_2,
      d1W_3, b1_3, d2W_3, b2_3)
    emb, rec1, rec2, rec3, hp1, hp2, hp3 = (x[:_N] for x in fuse_outs)

    return (h2_all[0, :_N], h2_all[1, :_N], h2_all[2, :_N], hp1, hp2, hp3,
            h2_all[3, :_N], h2_all[4, :_N], h2_all[5, :_N],
            emb, rec1, rec2, rec3)


# speedup vs baseline: 1.3236x; 1.3236x over previous
import jax
import jax.numpy as jnp
from jax import lax
from jax.experimental import pallas as pl
from jax.experimental.pallas import tpu as pltpu
from jax.experimental.pallas import tpu_sc as plsc

_N = 10000
_NP = 10240
_E = 320000
_DIN, _DH, _DO = 128, 64, 32
_NT = 16
_RPT = _N // _NT
_SEG = _NP // _NT
_CHK = 80
_NCH_G = (_E // _NT) // _CHK
_NCH_C = (_E // (2 * _NT)) // _CHK
_BLK = 2000
_BLKP = 2048


def _elu(x):
    return jnp.where(x > 0, x, jnp.exp(jnp.minimum(x, 0.0)) - 1.0)


def _prepass_body(f1, f2, f3, w1, w2, w3, a1, a2, a3,
                  z1, z2, z3, zs1, zd1, zs2, zd2, zs3, zd3):
    for f, w, a, z, zs, zd in ((f1, w1, a1, z1, zs1, zd1),
                               (f2, w2, a2, z2, zs2, zd2),
                               (f3, w3, a3, z3, zs3, zd3)):
        zb = jnp.dot(f[...], w[...], preferred_element_type=jnp.float32)
        z[...] = zb
        sd = jnp.dot(zb, a[...], preferred_element_type=jnp.float32)
        zs[...] = sd[:, 0:1]
        zd[...] = sd[:, 1:2]


def _h2_body(h1, w2, out):
    h = _elu(h1[0])
    out[0] = jnp.dot(h, w2[0], preferred_element_type=jnp.float32)


def _fuse_body(h2b, sums, cnts, w_om, u_om,
               d1W1, d1b1, d2W1, d2b1, d1W2, d1b2, d2W2, d2b2,
               d1W3, d1b3, d2W3, d2b3,
               emb_o, r1_o, r2_o, r3_o, hp1_o, hp2_o, hp3_o):
    h = (h2b[0], h2b[1], h2b[2])
    ex = []
    for c in range(3):
        v = jnp.tanh(jnp.dot(h[c], w_om[...], preferred_element_type=jnp.float32))
        vu = jnp.dot(v, u_om[...], preferred_element_type=jnp.float32)
        ex.append(jnp.exp(vu))
    denom = ex[0] + ex[1] + ex[2]
    emb = (h[0] * (ex[0] / denom) + h[1] * (ex[1] / denom)
           + h[2] * (ex[2] / denom))
    emb_o[...] = emb
    for dw1, db1, dw2, db2, ro in ((d1W1, d1b1, d2W1, d2b1, r1_o),
                                   (d1W2, d1b2, d2W2, d2b2, r2_o),
                                   (d1W3, d1b3, d2W3, d2b3, r3_o)):
        t = _elu(jnp.dot(emb, dw1[...], preferred_element_type=jnp.float32)
                       + db1[...])
        ro[...] = (jnp.dot(t, dw2[...], preferred_element_type=jnp.float32)
                   + db2[...])
    cnt3 = jnp.sum(cnts[...], axis=0).reshape(3, -1)
    s_all = sums[...]
    for m, hpo in enumerate((hp1_o, hp2_o, hp3_o)):
        sm = s_all[0, m] + s_all[1, m]
        hpo[...] = sm / jnp.maximum(cnt3[m], 1.0)[:, None]


def _zero_1d(ref, n):
    @pl.loop(0, n // 16)
    def _(i):
        ref[pl.ds(i * 16, 16)] = jnp.zeros((16,), jnp.float32)


def _zero_rows(ref, nrows, width):
    @pl.loop(0, nrows)
    def _(r):
        for c in range(width // 16):
            ref[r, pl.ds(c * 16, 16)] = jnp.zeros((16,), jnp.float32)


def _copy_rows_out(src_sh, dst, base):
    for q in range(_SEG // _CHK):
        pltpu.sync_copy(src_sh.at[pl.ds(base + q * _CHK, _CHK)],
                        dst.at[pl.ds(base + q * _CHK, _CHK)])


def _zero_sh_rows(zero_buf, dst_sh, base, sem=None):
    if sem is None:
        for q in range(_SEG // _CHK):
            pltpu.sync_copy(zero_buf, dst_sh.at[pl.ds(base + q * _CHK, _CHK)])
    else:
        for q in range(_SEG // _CHK):
            pltpu.async_copy(zero_buf, dst_sh.at[pl.ds(base + q * _CHK, _CHK)],
                             sem)
        for q in range(_SEG // _CHK):
            pltpu.make_async_copy(
                zero_buf, dst_sh.at[pl.ds(base + q * _CHK, _CHK)], sem).wait()


_SROW = _NP // 16
_SRQ = _SROW // 128


def _gat_body(ei1, ei2, ei3, eiC, zs1, zd1, zs2, zd2, zs3, zd3, z1, z2, z3,
              h1_out,
              src2, dst2, zs_t, zd_t, s_loc, rowsA, rowsB, rowsC, alpha_b,
              iota_r, zrow_b, sdiv, s_sum_sh, h1_sh,
              gsA, gsB, gsC, ssA, ssB, ssC):
    cid = lax.axis_index("c")
    sid = lax.axis_index("s")
    for q in range(_SRQ):
        for g in range(8):
            iota_r[q, pl.ds(g * 16, 16)] = (
                lax.iota(jnp.int32, 16) + (q * 128 + g * 16))
    _zero_rows(zrow_b, _SROW // _NT, 16)
    tabs = ((zs1, zd1, z1, ei1), (zs2, zd2, z2, ei2), (zs3, zd3, z3, ei3))
    for a in range(3):
        zs_h, zd_h, z_h, ei_h = tabs[a]

        @pl.when(cid == 0)
        def _():
            pltpu.sync_copy(ei_h.at[0, sid], src2)
            pltpu.sync_copy(ei_h.at[1, sid], dst2)

        if a == 0:
            @pl.when(cid == 1)
            def _():
                pltpu.sync_copy(eiC.at[0, sid], src2)
                pltpu.sync_copy(eiC.at[1, sid], dst2)

        pltpu.sync_copy(zs_h, zs_t)
        pltpu.sync_copy(zd_h, zd_t)
        _zero_rows(s_loc, _SROW, 16)
        _zero_rows(rowsA, _CHK, _DH)
        pltpu.sync_copy(zrow_b, s_sum_sh.at[pl.ds((_SROW // _NT) * sid,
                                                  _SROW // _NT)])
        _zero_sh_rows(rowsA, h1_sh, sid * _SEG, ssA)
        plsc.subcore_barrier()

        @pl.loop(0, _NCH_G, unroll=2)
        def _(j):
            for k in range(_CHK // 16):
                sv = src2[j, pl.ds(k * 16, 16)]
                dv = dst2[j, pl.ds(k * 16, 16)]
                e = plsc.load_gather(zs_t, [sv]) + plsc.load_gather(zd_t, [dv])
                e = jnp.maximum(e, 0.2 * e)
                exv = jnp.exp(e)
                plsc.addupdate_scatter(
                    s_loc, [lax.shift_right_logical(dv, 4),
                            lax.bitwise_and(dv, 15)], exv)

        for q in range(_SRQ):
            pltpu.sync_copy(s_loc.at[pl.ds(q * 128, 128)],
                            s_sum_sh.at[iota_r.at[q]], add=True)

        bufs = (rowsA, rowsB, rowsC)
        gsems = (gsA, gsB, gsC)
        ssems = (ssA, ssB, ssC)

        def _fire_g(j, b):
            pltpu.async_copy(z_h.at[src2.at[j]], bufs[b], gsems[b])

        def _wait_g(j, b):
            pltpu.make_async_copy(z_h.at[src2.at[j]], bufs[b],
                                  gsems[b]).wait()

        def _fire_s(j, b):
            pltpu.async_copy(bufs[b], h1_sh.at[dst2.at[j]], ssems[b],
                             add=True)

        def _wait_s(j, b):
            pltpu.make_async_copy(bufs[b], h1_sh.at[dst2.at[j]],
                                  ssems[b]).wait()

        def _scale(j, b):
            rows = bufs[b]
            for k in range(_CHK // 16):
                sv = src2[j, pl.ds(k * 16, 16)]
                dv = dst2[j, pl.ds(k * 16, 16)]
                e = plsc.load_gather(zs_t, [sv]) + plsc.load_gather(zd_t, [dv])
                e = jnp.maximum(e, 0.2 * e)
                alpha_b[pl.ds(k * 16, 16)] = jnp.exp(e)

            @pl.loop(0, _CHK, unroll=4)
            def _(r):
                av = plsc.load_gather(alpha_b, [jnp.full((16,), r, jnp.int32)])
                for c in range(_DH // 16):
                    rows[r, pl.ds(c * 16, 16)] = rows[r, pl.ds(c * 16, 16)] * av

        _fire_g(0, 0)
        _fire_g(1, 1)
        _wait_g(0, 0); _scale(0, 0); _fire_s(0, 0); _fire_g(2, 2)
        _wait_g(1, 1); _scale(1, 1); _fire_s(1, 1); _wait_s(0, 0)
        _fire_g(3, 0)

        @pl.loop(2, _NCH_G - 2, step=3)
        def _(t):
            for d in range(3):
                b = (2 + d) % 3
                _wait_g(t + d, b); _scale(t + d, b); _fire_s(t + d, b)
                _wait_s(t + d - 1, (b + 2) % 3); _fire_g(t + d + 2, (b + 2) % 3)

        _wait_g(_NCH_G - 2, 2); _scale(_NCH_G - 2, 2); _fire_s(_NCH_G - 2, 2)
        _wait_s(_NCH_G - 3, 1)
        _wait_g(_NCH_G - 1, 0); _scale(_NCH_G - 1, 0); _fire_s(_NCH_G - 1, 0)
        _wait_s(_NCH_G - 2, 2)
        _wait_s(_NCH_G - 1, 0)
        plsc.subcore_barrier()

        pltpu.sync_copy(s_sum_sh.at[pl.ds(sid * (_SEG // 16), _SEG // 16)],
                        sdiv)
        nq = _SEG // _CHK

        def _div_out(q, b):
            rows = bufs[b]
            rbase = sid * _SEG + q * _CHK

            @pl.loop(0, _CHK)
            def _(r):
                rq = r + q * _CHK
                sv = plsc.load_gather(
                    sdiv, [jnp.full((16,), lax.shift_right_logical(rq, 4),
                                    jnp.int32),
                           jnp.full((16,), lax.bitwise_and(rq, 15),
                                    jnp.int32)])
                inv = 1.0 / (sv + 1e-16)
                for c in range(_DH // 16):
                    rows[r, pl.ds(c * 16, 16)] = (
                        rows[r, pl.ds(c * 16, 16)] * inv)

            pltpu.sync_copy(rows, h1_out.at[cid, a, pl.ds(rbase, _CHK)])

        def _fire_in(q, b):
            pltpu.async_copy(h1_sh.at[pl.ds(sid * _SEG + q * _CHK, _CHK)],
                             bufs[b], gsems[b])

        def _wait_in(q, b):
            pltpu.make_async_copy(
                h1_sh.at[pl.ds(sid * _SEG + q * _CHK, _CHK)],
                bufs[b], gsems[b]).wait()

        _fire_in(0, 0)
        for q in range(nq):
            if q + 1 < nq:
                _fire_in(q + 1, (q + 1) % 3)
            _wait_in(q, q % 3)
            _div_out(q, q % 3)
        plsc.subcore_barrier()


def _csl_body(ei1, ei2, ei3, h2a,
              sums_out, cnt_out,
              gat2, sct2, cnt_loc, rows32A, rows32B, rows32C, iota_r, zrow_b,
              h2s_sh, cnt_sh,
              gsA, gsB, gsC, ssA, ssB, ssC):
    cid = lax.axis_index("c")
    sid = lax.axis_index("s")
    for q in range(_SRQ):
        for g in range(8):
            iota_r[q, pl.ds(g * 16, 16)] = (
                lax.iota(jnp.int32, 16) + (q * 128 + g * 16))
    _zero_rows(zrow_b, _SROW // _NT, 16)
    tabs = (ei1, ei2, ei3)
    for m in range(3):
        ei_h = tabs[m]
        h2_h = h2a.at[m]
        pltpu.sync_copy(ei_h.at[1, cid, sid], gat2)
        pltpu.sync_copy(ei_h.at[0, cid, sid], sct2)
        _zero_rows(cnt_loc, _SROW, 16)
        pltpu.sync_copy(zrow_b, cnt_sh.at[pl.ds((_SROW // _NT) * sid,
                                                _SROW // _NT)])

        @pl.loop(0, _NCH_C, unroll=2)
        def _(j):
            for k in range(_CHK // 16):
                sv = sct2[j, pl.ds(k * 16, 16)]
                plsc.addupdate_scatter(
                    cnt_loc, [lax.shift_right_logical(sv, 4),
                              lax.bitwise_and(sv, 15)],
                    jnp.ones((16,), jnp.float32))

        _zero_rows(rows32A, _CHK, _DO)
        _zero_sh_rows(rows32A, h2s_sh, sid * _SEG, ssA)
        plsc.subcore_barrier()
        for q in range(_SRQ):
            pltpu.sync_copy(cnt_loc.at[pl.ds(q * 128, 128)],
                            cnt_sh.at[iota_r.at[q]], add=True)

        bufs = (rows32A, rows32B, rows32C)
        gsems = (gsA, gsB, gsC)
        ssems = (ssA, ssB, ssC)

        def _fire_g(j, b):
            pltpu.async_copy(h2_h.at[gat2.at[j]], bufs[b], gsems[b])

        def _wait_g(j, b):
            pltpu.make_async_copy(h2_h.at[gat2.at[j]], bufs[b],
                                  gsems[b]).wait()

        def _fire_s(j, b):
            pltpu.async_copy(bufs[b], h2s_sh.at[sct2.at[j]], ssems[b],
                             add=True)

        def _wait_s(j, b):
            pltpu.make_async_copy(bufs[b], h2s_sh.at[sct2.at[j]],
                                  ssems[b]).wait()

        _fire_g(0, 0)
        _fire_g(1, 1)
        _wait_g(0, 0); _fire_s(0, 0); _fire_g(2, 2)
        _wait_g(1, 1); _fire_s(1, 1); _wait_s(0, 0); _fire_g(3, 0)

        @pl.loop(2, _NCH_C - 3, step=3)
        def _(t):
            for d in range(3):
                b = (2 + d) % 3
                _wait_g(t + d, b); _fire_s(t + d, b)
                _wait_s(t + d - 1, (b + 2) % 3)
                _fire_g(t + d + 2, (b + 2) % 3)

        _wait_g(122, 2); _fire_s(122, 2); _wait_s(121, 1); _fire_g(124, 1)
        _wait_g(123, 0); _fire_s(123, 0); _wait_s(122, 2)
        _wait_g(124, 1); _fire_s(124, 1); _wait_s(123, 0)
        _wait_s(124, 1)
        plsc.subcore_barrier()
        _copy_rows_out(h2s_sh, sums_out.at[cid, m], sid * _SEG)
        pltpu.sync_copy(cnt_sh.at[pl.ds((_SROW // _NT) * sid, _SROW // _NT)],
                        cnt_out.at[cid, m, pl.ds((_SROW // _NT) * sid,
                                                 _SROW // _NT)])
        plsc.subcore_barrier()


def kernel(features_1, features_2, features_3,
           edge_index_1, edge_index_2, edge_index_3, edge_CSL,
           W1_1, as_1, ad_1, W2_1, W1_2, as_2, ad_2, W2_2,
           W1_3, as_3, ad_3, W2_3, w_omega, u_omega,
           d1W_1, d1b_1, d2W_1, d2b_1, d1W_2, d1b_2, d2W_2, d2b_2,
           d1W_3, d1b_3, d2W_3, d2b_3):
    f32 = jnp.float32
    A1 = jnp.stack([as_1, ad_1], axis=1)
    A2 = jnp.stack([as_2, ad_2], axis=1)
    A3 = jnp.stack([as_3, ad_3], axis=1)

    nb = _N // _BLK
    outs = pl.pallas_call(
        _prepass_body,
        grid=(nb,),
        in_specs=[pl.BlockSpec((_BLK, _DIN), lambda i: (i, 0))] * 3
        + [pl.BlockSpec((_DIN, _DH), lambda i: (0, 0))] * 3
        + [pl.BlockSpec((_DH, 2), lambda i: (0, 0))] * 3,
        out_specs=[pl.BlockSpec((_BLK, _DH), lambda i: (i, 0))] * 3
        + [pl.BlockSpec((_BLK, 1), lambda i: (i, 0))] * 6,
        out_shape=[jax.ShapeDtypeStruct((_N, _DH), f32)] * 3
        + [jax.ShapeDtypeStruct((_N, 1), f32)] * 6,
    )(features_1, features_2, features_3, W1_1, W1_2, W1_3, A1, A2, A3)
    z1, z2, z3, zs1, zd1, zs2, zd2, zs3, zd3 = outs
    zs1, zd1, zs2, zd2, zs3, zd3 = (x.reshape(_N) for x in
                                    (zs1, zd1, zs2, zd2, zs3, zd3))

    eg = lambda e: e.reshape(2, _NT, _NCH_G, _CHK)
    ei1g, ei2g, ei3g, eiCg = (eg(e) for e in (edge_index_1, edge_index_2,
                                              edge_index_3, edge_CSL))
    h1_all = pl.kernel(
        _gat_body,
        out_type=jax.ShapeDtypeStruct((2, 3, _NP, _DH), f32),
        mesh=plsc.VectorSubcoreMesh(core_axis_name="c", subcore_axis_name="s"),
        compiler_params=pltpu.CompilerParams(needs_layout_passes=False, use_tc_tiling_on_sc=False),
        scratch_types=[
            pltpu.VMEM((_NCH_G, _CHK), jnp.int32),
            pltpu.VMEM((_NCH_G, _CHK), jnp.int32),
            pltpu.VMEM((_N,), f32),
            pltpu.VMEM((_N,), f32),
            pltpu.VMEM((_SROW, 16), f32),
            pltpu.VMEM((_CHK, _DH), f32),
            pltpu.VMEM((_CHK, _DH), f32),
            pltpu.VMEM((_CHK, _DH), f32),
            pltpu.VMEM((_CHK,), f32),
            pltpu.VMEM((_SRQ, 128), jnp.int32),
            pltpu.VMEM((_SROW // _NT, 16), f32),
            pltpu.VMEM((_SEG // 16, 16), f32),
            pltpu.VMEM_SHARED((_SROW, 16), f32),
            pltpu.VMEM_SHARED((_NP, _DH), f32),
        ] + [pltpu.SemaphoreType.DMA] * 6,
    )(ei1g, ei2g, ei3g, eiCg, zs1, zd1, zs2, zd2, zs3, zd3, z1, z2, z3)

    h1_flat = h1_all.reshape(6, _NP, _DH)
    W2s = jnp.stack([W2_1, W2_2, W2_3, W2_1, W2_2, W2_3])
    nbp = _NP // _BLKP
    h2_all = pl.pallas_call(
        _h2_body,
        grid=(6, nbp),
        in_specs=[pl.BlockSpec((1, _BLKP, _DH), lambda m, i: (m, i, 0)),
                  pl.BlockSpec((1, _DH, _DO), lambda m, i: (m, 0, 0))],
        out_specs=pl.BlockSpec((1, _BLKP, _DO), lambda m, i: (m, i, 0)),
        out_shape=jax.ShapeDtypeStruct((6, _NP, _DO), f32),
    )(h1_flat, W2s)

    ec = lambda e: e.reshape(2, 2, _NT, _NCH_C, _CHK)
    ei1c, ei2c, ei3c = (ec(e) for e in (edge_index_1, edge_index_2,
                                        edge_index_3))
    sums, cnts = pl.kernel(
        _csl_body,
        out_type=(jax.ShapeDtypeStruct((2, 3, _NP, _DO), f32),
                  jax.ShapeDtypeStruct((2, 3, _SROW, 16), f32)),
        mesh=plsc.VectorSubcoreMesh(core_axis_name="c", subcore_axis_name="s"),
        compiler_params=pltpu.CompilerParams(needs_layout_passes=False, use_tc_tiling_on_sc=False),
        scratch_types=[
            pltpu.VMEM((_NCH_C, _CHK), jnp.int32),
            pltpu.VMEM((_NCH_C, _CHK), jnp.int32),
            pltpu.VMEM((_SROW, 16), f32),
            pltpu.VMEM((_CHK, _DO), f32),
            pltpu.VMEM((_CHK, _DO), f32),
            pltpu.VMEM((_CHK, _DO), f32),
            pltpu.VMEM((_SRQ, 128), jnp.int32),
            pltpu.VMEM((_SROW // _NT, 16), f32),
            pltpu.VMEM_SHARED((_NP, _DO), f32),
            pltpu.VMEM_SHARED((_SROW, 16), f32),
        ] + [pltpu.SemaphoreType.DMA] * 6 + [
        ],
    )(ei1c, ei2c, ei3c, h2_all)

    b1_1, b2_1 = d1b_1.reshape(1, _DH), d2b_1.reshape(1, _DIN)
    b1_2, b2_2 = d1b_2.reshape(1, _DH), d2b_2.reshape(1, _DIN)
    b1_3, b2_3 = d1b_3.reshape(1, _DH), d2b_3.reshape(1, _DIN)
    u_om = u_omega.reshape(_DO, 1)
    fuse_outs = pl.pallas_call(
        _fuse_body,
        grid=(nbp,),
        in_specs=[pl.BlockSpec((3, _BLKP, _DO), lambda i: (0, i, 0)),
                  pl.BlockSpec((2, 3, _BLKP, _DO), lambda i: (0, 0, i, 0)),
                  pl.BlockSpec((2, 3, _BLKP // 16, 16),
                               lambda i: (0, 0, i, 0)),
                  pl.BlockSpec((_DO, _DO), lambda i: (0, 0)),
                  pl.BlockSpec((_DO, 1), lambda i: (0, 0))]
        + [pl.BlockSpec((_DO, _DH), lambda i: (0, 0)),
           pl.BlockSpec((1, _DH), lambda i: (0, 0)),
           pl.BlockSpec((_DH, _DIN), lambda i: (0, 0)),
           pl.BlockSpec((1, _DIN), lambda i: (0, 0))] * 3,
        out_specs=[pl.BlockSpec((_BLKP, _DO), lambda i: (i, 0))]
        + [pl.BlockSpec((_BLKP, _DIN), lambda i: (i, 0))] * 3
        + [pl.BlockSpec((_BLKP, _DO), lambda i: (i, 0))] * 3,
        out_shape=[jax.ShapeDtypeStruct((_NP, _DO), f32)]
        + [jax.ShapeDtypeStruct((_NP, _DIN), f32)] * 3
        + [jax.ShapeDtypeStruct((_NP, _DO), f32)] * 3,
    )(h2_all, sums, cnts, w_omega, u_om,
      d1W_1, b1_1, d2W_1, b2_1, d1W_2, b1_2, d2W_2, b2_2,
      d1W_3, b1_3, d2W_3, b2_3)
    emb, rec1, rec2, rec3, hp1, hp2, hp3 = (x[:_N] for x in fuse_outs)

    return (h2_all[0, :_N], h2_all[1, :_N], h2_all[2, :_N], hp1, hp2, hp3,
            h2_all[3, :_N], h2_all[4, :_N], h2_all[5, :_N],
            emb, rec1, rec2, rec3)
